# baseline (speedup 1.0000x reference)
import functools

import jax
import jax.numpy as jnp
from jax import lax
from jax.experimental import pallas as pl
from jax.experimental.pallas import tpu as pltpu
from jax.experimental.pallas import tpu_sc as plsc

B = 16384
K = 16
PACK = 8
WIDE = PACK * K

_info = plsc.get_sparse_core_info()
NC = _info.num_cores
NS = _info.num_subcores
NW = NC * NS
BPW = B // NW
NBUF = 8
OROW = BPW // PACK


def _gather_sc(uidx, iidx, wt, ht):
    mesh = plsc.VectorSubcoreMesh(core_axis_name="c", subcore_axis_name="s")

    scratch = (
        [pltpu.VMEM((BPW + K,), jnp.int32)] * 2
        + [pltpu.VMEM((K, WIDE), jnp.float32)] * (2 * NBUF)
        + [pltpu.VMEM((OROW, WIDE), jnp.float32)] * 2
        + [pltpu.SemaphoreType.DMA] * (2 * NBUF)
    )

    @functools.partial(
        pl.kernel,
        mesh=mesh,
        compiler_params=pltpu.CompilerParams(needs_layout_passes=False),
        out_type=(
            jax.ShapeDtypeStruct((B // PACK, WIDE), jnp.float32),
            jax.ShapeDtypeStruct((B // PACK, WIDE), jnp.float32),
        ),
        scratch_types=scratch,
    )
    def k(uidx_hbm, iidx_hbm, w_hbm, h_hbm,
          u_out, v_out, *sc):
        uidx_v, iidx_v = sc[0:2]
        ubufs = sc[2:2 + NBUF]
        vbufs = sc[2 + NBUF:2 + 2 * NBUF]
        upack, vpack = sc[2 + 2 * NBUF:4 + 2 * NBUF]
        usems = sc[4 + 2 * NBUF:4 + 3 * NBUF]
        vsems = sc[4 + 3 * NBUF:4 + 4 * NBUF]

        wid = lax.axis_index("s") * NC + lax.axis_index("c")
        base = wid * BPW
        pltpu.sync_copy(uidx_hbm.at[pl.ds(base, BPW)],
                        uidx_v.at[pl.ds(0, BPW)])
        pltpu.sync_copy(iidx_hbm.at[pl.ds(base, BPW)],
                        iidx_v.at[pl.ds(0, BPW)])

        iota = lax.iota(jnp.int32, K)

        def fire(tbl, idx_v, buf, sem, r):
            raw = idx_v[pl.ds(r, K)][0]
            s = pl.multiple_of(lax.shift_right_logical(raw, 7) * WIDE, WIDE)
            pltpu.async_copy(tbl.at[:, pl.ds(s, WIDE)], buf, sem)

        def wait(tbl, buf, sem):
            pltpu.make_async_copy(tbl.at[:, pl.ds(0, WIDE)], buf, sem).wait()

        def extract(buf, idx_v, pack_buf, r):
            col = plsc.load_gather(idx_v, [jnp.full((K,), r, jnp.int32)]) & 127
            vals = plsc.load_gather(buf, [iota, col])
            prow = jnp.full((K,), lax.shift_right_logical(r, 3), jnp.int32)
            pcol = (r & 7) * K + iota
            plsc.store_scatter(pack_buf, [prow, pcol], vals)

        for b in range(NBUF):
            fire(w_hbm, uidx_v, ubufs[b], usems[b], b)
            fire(h_hbm, iidx_v, vbufs[b], vsems[b], b)

        def body(t, carry):
            for b in range(NBUF):
                r = t * NBUF + b
                wait(w_hbm, ubufs[b], usems[b])
                extract(ubufs[b], uidx_v, upack, r)
                fire(w_hbm, uidx_v, ubufs[b], usems[b], r + NBUF)
                wait(h_hbm, vbufs[b], vsems[b])
                extract(vbufs[b], iidx_v, vpack, r)
                fire(h_hbm, iidx_v, vbufs[b], vsems[b], r + NBUF)
            return carry

        lax.fori_loop(0, BPW // NBUF - 1, body, 0)

        for b in range(NBUF):
            r = BPW - NBUF + b
            wait(w_hbm, ubufs[b], usems[b])
            extract(ubufs[b], uidx_v, upack, r)
            wait(h_hbm, vbufs[b], vsems[b])
            extract(vbufs[b], iidx_v, vpack, r)

        pltpu.sync_copy(upack, u_out.at[pl.ds(wid * OROW, OROW)])
        pltpu.sync_copy(vpack, v_out.at[pl.ds(wid * OROW, OROW)])

    return k(uidx, iidx, wt, ht)


def _mlp_body(u_ref, v_ref, a_ref, b_ref, b1_ref, w2_ref, o_ref):
    h = jnp.dot(u_ref[...], a_ref[...], preferred_element_type=jnp.float32)
    h = h + jnp.dot(v_ref[...], b_ref[...], preferred_element_type=jnp.float32)
    h = jnp.maximum(h + b1_ref[...], 0.0)
    o_ref[...] = jnp.dot(h, w2_ref[...], preferred_element_type=jnp.float32)


def _mlp_tc(u, v, a, bm, b1t, w2b):
    return pl.pallas_call(
        _mlp_body,
        out_shape=jax.ShapeDtypeStruct((B // PACK, PACK), jnp.float32),
    )(u, v, a, bm, b1t, w2b)


def kernel(x, W_table, H_table, W_r_table, H_r_table, W1, b1, W2):
    uidx = x[:, 0]
    iidx = x[:, 1]
    wt = W_table.T
    ht = H_table.T
    u, v = _gather_sc(uidx, iidx, wt, ht)

    eye = jnp.eye(PACK, dtype=jnp.float32)
    a = jnp.kron(eye, W1[:, :K].T)
    bm = jnp.kron(eye, W1[:, K:].T)
    b1t = jnp.tile(b1, PACK).reshape(1, WIDE)
    w2b = jnp.kron(eye, W2.reshape(K, 1))
    out = _mlp_tc(u, v, a, bm, b1t, w2b)
    return out.reshape(B, 1)
